# Initial kernel scaffold; baseline (speedup 1.0000x reference)
import jax
import jax.numpy as jnp
from jax.experimental import pallas as pl


def kernel(modality_tokens, timestamps, channel_embed, pos_embed, month_table):
    raise NotImplementedError("write your pallas kernel here")


# speedup vs baseline: 1.4026x; 1.4026x over previous
import jax
import jax.numpy as jnp
from jax.experimental import pallas as pl
from jax.experimental.pallas import tpu as pltpu

B, H, W, T, BS, EMBED = 2, 16, 16, 12, 8, 1024
N = EMBED // 4
ROWS_PER_B = H * W * T * BS
PERIOD = T * BS
R = 4


def _encoder_body(months_ref,
                  tokens_ref,
                  channel_ref,
                  pos_ref,
                  month_ref,
                  out_ref,
                  addend_ref):
    i = pl.program_id(0)

    @pl.when(i == 0)
    def _build_addend():
        for b in range(B):
            base = b * PERIOD
            for t in range(T):
                row0 = base + t * BS
                addend_ref[pl.ds(row0, BS), 0:N] = channel_ref[...]
                addend_ref[pl.ds(row0, BS), N:2 * N] = jnp.broadcast_to(
                    pos_ref[t, :][None, :], (BS, N))
                m = months_ref[b * T + t]
                addend_ref[pl.ds(row0, BS), 2 * N:3 * N] = jnp.broadcast_to(
                    month_ref[m, :][None, :], (BS, N))
                addend_ref[pl.ds(row0, BS), 3 * N:] = jnp.zeros(
                    (BS, N), jnp.float32)

    steps_per_b = ROWS_PER_B // (R * PERIOD)
    b = i // steps_per_b
    add = addend_ref[pl.ds(b * PERIOD, PERIOD), :]
    out_ref[...] = tokens_ref[...] + add[None, :, :]


@jax.jit
def kernel(modality_tokens, timestamps, channel_embed, pos_embed, month_table):
    months = timestamps[:, :, 1].reshape(-1).astype(jnp.int32)
    tokens = modality_tokens.reshape(-1, PERIOD, EMBED)
    num_blocks = tokens.shape[0] // R

    grid_spec = pltpu.PrefetchScalarGridSpec(
        num_scalar_prefetch=1,
        grid=(num_blocks,),
        in_specs=[
            pl.BlockSpec((R, PERIOD, EMBED), lambda i, m: (i, 0, 0)),
            pl.BlockSpec((BS, N), lambda i, m: (0, 0)),
            pl.BlockSpec((T, N), lambda i, m: (0, 0)),
            pl.BlockSpec((12, N), lambda i, m: (0, 0)),
        ],
        out_specs=pl.BlockSpec((R, PERIOD, EMBED), lambda i, m: (i, 0, 0)),
        scratch_shapes=[pltpu.VMEM((B * PERIOD, EMBED), jnp.float32)],
    )

    out = pl.pallas_call(
        _encoder_body,
        grid_spec=grid_spec,
        out_shape=jax.ShapeDtypeStruct(tokens.shape, jnp.float32),
    )(months, tokens, channel_embed, pos_embed[:T], month_table)
    return out.reshape(B, H, W, T, BS, EMBED)


# speedup vs baseline: 1.6813x; 1.1986x over previous
import jax
import jax.numpy as jnp
from jax.experimental import pallas as pl
from jax.experimental.pallas import tpu as pltpu

B, H, W, T, BS, EMBED = 2, 16, 16, 12, 8, 1024
N = EMBED // 4
ROWS_PER_B = H * W * T * BS
PERIOD = T * BS
R = 8


def _encoder_body(months_ref,
                  tokens_ref,
                  channel_ref,
                  pos_ref,
                  month_ref,
                  out_ref,
                  addend_ref):
    i = pl.program_id(0)

    @pl.when(i == 0)
    def _build_addend():
        for b in range(B):
            base = b * PERIOD
            for t in range(T):
                row0 = base + t * BS
                addend_ref[pl.ds(row0, BS), 0:N] = channel_ref[...]
                addend_ref[pl.ds(row0, BS), N:2 * N] = jnp.broadcast_to(
                    pos_ref[t, :][None, :], (BS, N))
                m = months_ref[b * T + t]
                addend_ref[pl.ds(row0, BS), 2 * N:3 * N] = jnp.broadcast_to(
                    month_ref[m, :][None, :], (BS, N))
                addend_ref[pl.ds(row0, BS), 3 * N:] = jnp.zeros(
                    (BS, N), jnp.float32)

    steps_per_b = ROWS_PER_B // (R * PERIOD)
    b = i // steps_per_b
    add = addend_ref[pl.ds(b * PERIOD, PERIOD), :]
    out_ref[...] = tokens_ref[...] + add[None, :, :]


@jax.jit
def kernel(modality_tokens, timestamps, channel_embed, pos_embed, month_table):
    months = timestamps[:, :, 1].reshape(-1).astype(jnp.int32)
    tokens = modality_tokens.reshape(-1, PERIOD, EMBED)
    num_blocks = tokens.shape[0] // R

    grid_spec = pltpu.PrefetchScalarGridSpec(
        num_scalar_prefetch=1,
        grid=(num_blocks,),
        in_specs=[
            pl.BlockSpec((R, PERIOD, EMBED), lambda i, m: (i, 0, 0)),
            pl.BlockSpec((BS, N), lambda i, m: (0, 0)),
            pl.BlockSpec((T, N), lambda i, m: (0, 0)),
            pl.BlockSpec((12, N), lambda i, m: (0, 0)),
        ],
        out_specs=pl.BlockSpec((R, PERIOD, EMBED), lambda i, m: (i, 0, 0)),
        scratch_shapes=[pltpu.VMEM((B * PERIOD, EMBED), jnp.float32)],
    )

    out = pl.pallas_call(
        _encoder_body,
        grid_spec=grid_spec,
        out_shape=jax.ShapeDtypeStruct(tokens.shape, jnp.float32),
    )(months, tokens, channel_embed, pos_embed[:T], month_table)
    return out.reshape(B, H, W, T, BS, EMBED)


# speedup vs baseline: 1.7371x; 1.0332x over previous
import jax
import jax.numpy as jnp
from jax.experimental import pallas as pl
from jax.experimental.pallas import tpu as pltpu

B, H, W, T, BS, EMBED = 2, 16, 16, 12, 8, 1024
N = EMBED // 4
ROWS_PER_B = H * W * T * BS
PERIOD = T * BS
R = 16


def _encoder_body(months_ref,
                  tokens_ref,
                  channel_ref,
                  pos_ref,
                  month_ref,
                  out_ref,
                  addend_ref):
    i = pl.program_id(0)

    @pl.when(i == 0)
    def _build_addend():
        for b in range(B):
            base = b * PERIOD
            for t in range(T):
                row0 = base + t * BS
                addend_ref[pl.ds(row0, BS), 0:N] = channel_ref[...]
                addend_ref[pl.ds(row0, BS), N:2 * N] = jnp.broadcast_to(
                    pos_ref[t, :][None, :], (BS, N))
                m = months_ref[b * T + t]
                addend_ref[pl.ds(row0, BS), 2 * N:3 * N] = jnp.broadcast_to(
                    month_ref[m, :][None, :], (BS, N))
                addend_ref[pl.ds(row0, BS), 3 * N:] = jnp.zeros(
                    (BS, N), jnp.float32)

    steps_per_b = ROWS_PER_B // (R * PERIOD)
    b = i // steps_per_b
    add = addend_ref[pl.ds(b * PERIOD, PERIOD), :]
    out_ref[...] = tokens_ref[...] + add[None, :, :]


@jax.jit
def kernel(modality_tokens, timestamps, channel_embed, pos_embed, month_table):
    months = timestamps[:, :, 1].reshape(-1).astype(jnp.int32)
    tokens = modality_tokens.reshape(-1, PERIOD, EMBED)
    num_blocks = tokens.shape[0] // R

    grid_spec = pltpu.PrefetchScalarGridSpec(
        num_scalar_prefetch=1,
        grid=(num_blocks,),
        in_specs=[
            pl.BlockSpec((R, PERIOD, EMBED), lambda i, m: (i, 0, 0)),
            pl.BlockSpec((BS, N), lambda i, m: (0, 0)),
            pl.BlockSpec((T, N), lambda i, m: (0, 0)),
            pl.BlockSpec((12, N), lambda i, m: (0, 0)),
        ],
        out_specs=pl.BlockSpec((R, PERIOD, EMBED), lambda i, m: (i, 0, 0)),
        scratch_shapes=[pltpu.VMEM((B * PERIOD, EMBED), jnp.float32)],
    )

    out = pl.pallas_call(
        _encoder_body,
        grid_spec=grid_spec,
        out_shape=jax.ShapeDtypeStruct(tokens.shape, jnp.float32),
    )(months, tokens, channel_embed, pos_embed[:T], month_table)
    return out.reshape(B, H, W, T, BS, EMBED)


# speedup vs baseline: 1.7537x; 1.0095x over previous
import jax
import jax.numpy as jnp
from jax.experimental import pallas as pl
from jax.experimental.pallas import tpu as pltpu

B, H, W, T, BS, EMBED = 2, 16, 16, 12, 8, 1024
N = EMBED // 4
ROWS_PER_B = H * W * T * BS
PERIOD = T * BS
R = 32


def _encoder_body(months_ref,
                  tokens_ref,
                  channel_ref,
                  pos_ref,
                  month_ref,
                  out_ref,
                  addend_ref):
    i = pl.program_id(0)

    @pl.when(i == 0)
    def _build_addend():
        for b in range(B):
            base = b * PERIOD
            for t in range(T):
                row0 = base + t * BS
                addend_ref[pl.ds(row0, BS), 0:N] = channel_ref[...]
                addend_ref[pl.ds(row0, BS), N:2 * N] = jnp.broadcast_to(
                    pos_ref[t, :][None, :], (BS, N))
                m = months_ref[b * T + t]
                addend_ref[pl.ds(row0, BS), 2 * N:3 * N] = jnp.broadcast_to(
                    month_ref[m, :][None, :], (BS, N))
                addend_ref[pl.ds(row0, BS), 3 * N:] = jnp.zeros(
                    (BS, N), jnp.float32)

    steps_per_b = ROWS_PER_B // (R * PERIOD)
    b = i // steps_per_b
    add = addend_ref[pl.ds(b * PERIOD, PERIOD), :]
    out_ref[...] = tokens_ref[...] + add[None, :, :]


@jax.jit
def kernel(modality_tokens, timestamps, channel_embed, pos_embed, month_table):
    months = timestamps[:, :, 1].reshape(-1).astype(jnp.int32)
    tokens = modality_tokens.reshape(-1, PERIOD, EMBED)
    num_blocks = tokens.shape[0] // R

    grid_spec = pltpu.PrefetchScalarGridSpec(
        num_scalar_prefetch=1,
        grid=(num_blocks,),
        in_specs=[
            pl.BlockSpec((R, PERIOD, EMBED), lambda i, m: (i, 0, 0)),
            pl.BlockSpec((BS, N), lambda i, m: (0, 0)),
            pl.BlockSpec((T, N), lambda i, m: (0, 0)),
            pl.BlockSpec((12, N), lambda i, m: (0, 0)),
        ],
        out_specs=pl.BlockSpec((R, PERIOD, EMBED), lambda i, m: (i, 0, 0)),
        scratch_shapes=[pltpu.VMEM((B * PERIOD, EMBED), jnp.float32)],
    )

    out = pl.pallas_call(
        _encoder_body,
        grid_spec=grid_spec,
        out_shape=jax.ShapeDtypeStruct(tokens.shape, jnp.float32),
    )(months, tokens, channel_embed, pos_embed[:T], month_table)
    return out.reshape(B, H, W, T, BS, EMBED)
